# baseline (speedup 1.0000x reference)
import functools

import jax
import jax.numpy as jnp
from jax import lax
from jax.experimental import pallas as pl
from jax.experimental.pallas import tpu as pltpu
from jax.experimental.pallas import tpu_sc as plsc

D = 768
HD = D // 2
V = 1024
G = 32
B = 16384
NC = 2
NS = 16
NW = NC * NS
B_PER_W = B // NW
CHUNK = 64
NBUF = 2
NCHUNK = B_PER_W // CHUNK
LANES = 16


def _body(wtab_hbm, htab_hbm, idx_hbm, out_hbm,
          wtab_v, htab_v, idx_v, woff_v, hoff_v, stage_v, ssems):
    cid = lax.axis_index("c")
    sid = lax.axis_index("s")
    wid = sid * NC + cid
    base = wid * B_PER_W

    pltpu.sync_copy(idx_hbm.at[pl.ds(base, B_PER_W)], idx_v)
    pltpu.sync_copy(wtab_hbm, wtab_v)
    pltpu.sync_copy(htab_hbm, htab_v)

    for j in range(B_PER_W // LANES):
        v = idx_v[pl.ds(j * LANES, LANES)]
        woff_v[pl.ds(j * LANES, LANES)] = lax.shift_right_logical(v, 5) * HD
        hoff_v[pl.ds(j * LANES, LANES)] = lax.bitwise_and(v, G - 1) * HD

    def assemble(c, buf):
        @plsc.parallel_loop(0, CHUNK, unroll=4)
        def row(r):
            wo = woff_v[pl.ds(c * CHUNK + r, LANES)][0]
            ho = hoff_v[pl.ds(c * CHUNK + r, LANES)][0]
            rb = buf * CHUNK * D + r * D
            for k in range(HD // LANES):
                stage_v[pl.ds(rb + k * LANES, LANES)] = (
                    wtab_v[pl.ds(wo + k * LANES, LANES)])
            for k in range(HD // LANES):
                stage_v[pl.ds(rb + HD + k * LANES, LANES)] = (
                    htab_v[pl.ds(ho + k * LANES, LANES)])

    def store(c):
        return pltpu.async_copy(
            stage_v.at[pl.ds((c % NBUF) * CHUNK * D, CHUNK * D)],
            out_hbm.at[pl.ds((base + c * CHUNK) * D, CHUNK * D)],
            ssems.at[c % NBUF],
        )

    scp = [None] * NCHUNK
    for c in range(NCHUNK):
        if c >= NBUF:
            scp[c - NBUF].wait()
        assemble(c, c % NBUF)
        scp[c] = store(c)
    for c in range(NCHUNK - NBUF, NCHUNK):
        scp[c].wait()


@jax.jit
def _gather(pos, pos_embd):
    wtab = pos_embd[::G, :HD]
    htab = pos_embd[:G, HD:]
    mesh = plsc.VectorSubcoreMesh(core_axis_name="c", subcore_axis_name="s")
    run = pl.kernel(
        _body,
        mesh=mesh,
        compiler_params=pltpu.CompilerParams(needs_layout_passes=False),
        out_type=jax.ShapeDtypeStruct((B * D,), jnp.float32),
        scratch_types=[
            pltpu.VMEM((G * HD,), jnp.float32),
            pltpu.VMEM((G * HD,), jnp.float32),
            pltpu.VMEM((B_PER_W,), jnp.int32),
            pltpu.VMEM((B_PER_W + LANES,), jnp.int32),
            pltpu.VMEM((B_PER_W + LANES,), jnp.int32),
            pltpu.VMEM((NBUF * CHUNK * D,), jnp.float32),
            pltpu.SemaphoreType.DMA((NBUF,)),
        ],
    )
    out = run(jnp.reshape(wtab, (G * HD,)), jnp.reshape(htab, (G * HD,)), pos)
    return jnp.reshape(out, (B, D))


def kernel(pos, pos_embd):
    return _gather(pos, pos_embd)


# speedup vs baseline: 1.1327x; 1.1327x over previous
import functools

import jax
import jax.numpy as jnp
from jax import lax
from jax.experimental import pallas as pl
from jax.experimental.pallas import tpu as pltpu
from jax.experimental.pallas import tpu_sc as plsc

D = 768
HD = D // 2
V = 1024
G = 32
B = 16384
NC = 2
NS = 16
NW = NC * NS
B_PER_W = B // NW
CHUNK = 64
NBUF = 2
NCHUNK = B_PER_W // CHUNK
LANES = 16


def _body(wtab_hbm, htab_hbm, idx_hbm, out_hbm,
          wtab_v, htab_v, idx_v, woff_v, hoff_v, stage_v, ssems):
    cid = lax.axis_index("c")
    sid = lax.axis_index("s")
    wid = sid * NC + cid
    base = wid * B_PER_W

    pltpu.sync_copy(idx_hbm.at[pl.ds(base, B_PER_W)], idx_v)
    pltpu.sync_copy(wtab_hbm, wtab_v)
    pltpu.sync_copy(htab_hbm, htab_v)

    for j in range(B_PER_W // LANES):
        v = idx_v[pl.ds(j * LANES, LANES)]
        woff_v[pl.ds(j * LANES, LANES)] = lax.shift_right_logical(v, 5) * HD
        hoff_v[pl.ds(j * LANES, LANES)] = lax.bitwise_and(v, G - 1) * HD

    def drain_one():
        pltpu.make_async_copy(
            out_hbm.at[pl.ds(0, CHUNK * D)],
            stage_v.at[pl.ds(0, CHUNK * D)],
            ssems,
        ).wait()

    def assemble(buf, c):
        @plsc.parallel_loop(0, CHUNK, unroll=4)
        def row(r):
            wo = woff_v[pl.ds(c * CHUNK + r, LANES)][0]
            ho = hoff_v[pl.ds(c * CHUNK + r, LANES)][0]
            rb = buf * CHUNK * D + r * D
            for k in range(HD // LANES):
                stage_v[pl.ds(rb + k * LANES, LANES)] = (
                    wtab_v[pl.ds(wo + k * LANES, LANES)])
            for k in range(HD // LANES):
                stage_v[pl.ds(rb + HD + k * LANES, LANES)] = (
                    htab_v[pl.ds(ho + k * LANES, LANES)])

    def chunk(c, carry):
        buf = lax.rem(c, NBUF)

        @pl.when(c >= NBUF)
        def _():
            drain_one()

        for b in range(NBUF):
            @pl.when(buf == b)
            def _(b=b):
                assemble(b, c)

        pltpu.async_copy(
            stage_v.at[pl.ds(buf * CHUNK * D, CHUNK * D)],
            out_hbm.at[pl.ds((base + c * CHUNK) * D, CHUNK * D)],
            ssems,
        )
        return carry

    lax.fori_loop(0, NCHUNK, chunk, 0)
    for _ in range(NBUF):
        drain_one()


@jax.jit
def _gather(pos, pos_embd):
    wtab = pos_embd[::G, :HD]
    htab = pos_embd[:G, HD:]
    mesh = plsc.VectorSubcoreMesh(core_axis_name="c", subcore_axis_name="s")
    run = pl.kernel(
        _body,
        mesh=mesh,
        compiler_params=pltpu.CompilerParams(needs_layout_passes=False),
        out_type=jax.ShapeDtypeStruct((B * D,), jnp.float32),
        scratch_types=[
            pltpu.VMEM((G * HD,), jnp.float32),
            pltpu.VMEM((G * HD,), jnp.float32),
            pltpu.VMEM((B_PER_W,), jnp.int32),
            pltpu.VMEM((B_PER_W + LANES,), jnp.int32),
            pltpu.VMEM((B_PER_W + LANES,), jnp.int32),
            pltpu.VMEM((NBUF * CHUNK * D,), jnp.float32),
            pltpu.SemaphoreType.DMA,
        ],
    )
    out = run(jnp.reshape(wtab, (G * HD,)), jnp.reshape(htab, (G * HD,)), pos)
    return jnp.reshape(out, (B, D))


def kernel(pos, pos_embd):
    return _gather(pos, pos_embd)


# speedup vs baseline: 1.8211x; 1.6078x over previous
import functools

import jax
import jax.numpy as jnp
from jax import lax
from jax.experimental import pallas as pl
from jax.experimental.pallas import tpu as pltpu
from jax.experimental.pallas import tpu_sc as plsc

D = 768
V = 1024
B = 16384
NC = 2
NS = 16
NW = NC * NS
B_PER_W = B // NW
CHUNK = 16
NBUF = 8
NCHUNK = B_PER_W // CHUNK
R = 8


def _gather_body(table_hbm, idx_hbm, out_hbm, idx_v, rows_v, gsems, ssems):
    cid = lax.axis_index("c")
    sid = lax.axis_index("s")
    wid = sid * NC + cid
    base = wid * B_PER_W

    pltpu.sync_copy(idx_hbm.at[pl.ds(base, B_PER_W)], idx_v)

    def gather(c):
        return pltpu.async_copy(
            table_hbm.at[idx_v.at[pl.ds(c * CHUNK, CHUNK)]],
            rows_v.at[c % NBUF],
            gsems.at[c % NBUF],
        )

    def store(c):
        return pltpu.async_copy(
            rows_v.at[c % NBUF],
            out_hbm.at[pl.ds(base + c * CHUNK, CHUNK)],
            ssems.at[c % NBUF],
        )

    skew = NBUF // 2
    gcp = [None] * NCHUNK
    scp = [None] * NCHUNK
    for i in range(skew):
        gcp[i] = gather(i)
    for i in range(NCHUNK):
        if i >= skew:
            scp[i - skew].wait()
        nxt = i + skew
        if nxt < NCHUNK:
            gcp[nxt] = gather(nxt)
        gcp[i].wait()
        scp[i] = store(i)
    for i in range(NCHUNK - skew, NCHUNK):
        scp[i].wait()


@jax.jit
def _gather(pos, pos_embd):
    mesh = plsc.VectorSubcoreMesh(core_axis_name="c", subcore_axis_name="s")
    run = pl.kernel(
        _gather_body,
        mesh=mesh,
        out_type=jax.ShapeDtypeStruct((B, D), jnp.float32),
        scratch_types=[
            pltpu.VMEM((B_PER_W,), jnp.int32),
            pltpu.VMEM((NBUF, CHUNK, D), jnp.float32),
            pltpu.SemaphoreType.DMA((NBUF,)),
            pltpu.SemaphoreType.DMA((NBUF,)),
        ],
    )
    return run(pos_embd, pos)


def kernel(pos, pos_embd):
    return _gather(pos, pos_embd)
